# baseline (speedup 1.0000x reference)
import functools

import jax
import jax.numpy as jnp
from jax.experimental import pallas as pl
from jax.experimental.pallas import tpu as pltpu


def _mil_body(nb, bn, h_ref, w1_ref, b1_ref, wb_ref, bb_ref, wc_ref,
              bc_ref, wcls_ref, bcls_ref,
              araw_ref, logits_ref, yprob_ref, yhat_ref,
              bestv_ref, bestf_ref, w1t_ref, wbt_ref, arows_ref):
    i = pl.program_id(0)

    @pl.when(i == 0)
    def _init():
        bestv_ref[0, 0] = -jnp.inf
        w1t_ref[...] = w1_ref[...].astype(w1t_ref.dtype).T
        wbt_ref[...] = wb_ref[...].astype(wbt_ref.dtype).T

    x = jnp.dot(h_ref[...].astype(w1t_ref.dtype), w1t_ref[...],
                preferred_element_type=jnp.float32)
    x = jnp.maximum(x + b1_ref[...], 0.0)
    gate = jnp.dot(x.astype(wbt_ref.dtype), wbt_ref[...],
                   preferred_element_type=jnp.float32)
    gate = jax.nn.sigmoid(gate + bb_ref[...])
    feat = x * gate
    a = jnp.sum(feat * wc_ref[...], axis=1, keepdims=True)
    a = a + bc_ref[0, 0]
    arows_ref[i, :, :] = a.T

    av = a[:, 0]
    bmax = jnp.max(av)
    bidx = jnp.argmax(av)

    @pl.when(bmax > bestv_ref[0, 0])
    def _update():
        bestv_ref[0, 0] = bmax
        rows = jax.lax.broadcasted_iota(jnp.int32, feat.shape, 0)
        bestf_ref[...] = jnp.sum(
            jnp.where(rows == bidx, feat, 0.0), axis=0, keepdims=True)

    @pl.when(i == nb - 1)
    def _finish():
        m = bestf_ref[...]
        logits = jax.lax.dot_general(
            m, wcls_ref[...], (((1,), (1,)), ((), ())),
            preferred_element_type=jnp.float32) + bcls_ref[...]
        logits_ref[...] = logits
        yprob_ref[...] = jax.nn.softmax(logits, axis=1)
        yhat_ref[...] = jnp.argmax(logits, axis=1).reshape(1, 1).astype(jnp.int32)
        araw_ref[...] = jnp.concatenate(
            [arows_ref[k, :, :] for k in range(nb)], axis=1)


@jax.jit
def kernel(h, W1, b1, Wb, bb, Wc, bc, Wcls, bcls):
    N, E = h.shape
    H = W1.shape[0]
    n_classes = Wcls.shape[0]

    bn = 5000 if N % 5000 == 0 else None
    if bn is None:
        for cand in (2000, 1000, 500, 400, 250, 200, 125, 100, 50, 25, 8, 1):
            if N % cand == 0:
                bn = cand
                break
    nb = N // bn

    b1r = b1.reshape(1, H)
    bbr = bb.reshape(1, H)
    wcr = Wc.reshape(1, H)
    bcr = bc.reshape(1, 1)
    bclsr = bcls.reshape(1, n_classes)

    araw, logits, yprob, yhat = pl.pallas_call(
        functools.partial(_mil_body, nb, bn),
        grid=(nb,),
        in_specs=[
            pl.BlockSpec((bn, E), lambda i: (i, 0)),
            pl.BlockSpec((H, E), lambda i: (0, 0)),
            pl.BlockSpec((1, H), lambda i: (0, 0)),
            pl.BlockSpec((H, H), lambda i: (0, 0)),
            pl.BlockSpec((1, H), lambda i: (0, 0)),
            pl.BlockSpec((1, H), lambda i: (0, 0)),
            pl.BlockSpec((1, 1), lambda i: (0, 0)),
            pl.BlockSpec((n_classes, H), lambda i: (0, 0)),
            pl.BlockSpec((1, n_classes), lambda i: (0, 0)),
        ],
        out_specs=[
            pl.BlockSpec((1, N), lambda i: (0, 0)),
            pl.BlockSpec((1, n_classes), lambda i: (0, 0)),
            pl.BlockSpec((1, n_classes), lambda i: (0, 0)),
            pl.BlockSpec((1, 1), lambda i: (0, 0)),
        ],
        out_shape=[
            jax.ShapeDtypeStruct((1, N), jnp.float32),
            jax.ShapeDtypeStruct((1, n_classes), jnp.float32),
            jax.ShapeDtypeStruct((1, n_classes), jnp.float32),
            jax.ShapeDtypeStruct((1, 1), jnp.int32),
        ],
        scratch_shapes=[
            pltpu.SMEM((1, 1), jnp.float32),
            pltpu.VMEM((1, H), jnp.float32),
            pltpu.VMEM((E, H), jnp.bfloat16),
            pltpu.VMEM((H, H), jnp.bfloat16),
            pltpu.VMEM((nb, 1, bn), jnp.float32),
        ],
        compiler_params=pltpu.CompilerParams(
            dimension_semantics=("arbitrary",),
        ),
    )(h, W1, b1r, Wb, bbr, wcr, bcr, Wcls, bclsr)

    return logits, yprob, yhat, araw
